# baseline (speedup 1.0000x reference)
import functools

import jax
import jax.numpy as jnp
from jax import lax
from jax.experimental import pallas as pl
from jax.experimental.pallas import tpu as pltpu
from jax.experimental.pallas import tpu_sc as plsc

B = 4
V1 = 1024
V2 = 256
V3 = 64
S = 7
NBR = 10
F32 = jnp.float32


def _silu(x):
    return x / (1.0 + jnp.exp(-x))


def _normcols(d):
    n = jnp.sqrt(jnp.sum(d * d, axis=0, keepdims=True))
    return d / jnp.maximum(n, 1e-12)


def _knn_body(x_ref, out_ref, *, V, K):
    b = pl.program_id(0)
    x = x_ref[0]
    q = jnp.sum(x * x, axis=1)
    d = (q[:, None] + q[None, :]
         - 2.0 * lax.dot_general(x, x, (((1,), (1,)), ((), ())),
                                 preferred_element_type=F32))
    iota = lax.broadcasted_iota(jnp.int32, (V, V), 1)
    out_ref[0] = jnp.zeros((16, V), jnp.int32)
    for p in range(K + 1):
        m = jnp.min(d, axis=1)
        eq = d == m[:, None]
        j = jnp.min(jnp.where(eq, iota, V), axis=1)
        if p > 0:
            out_ref[0, p - 1, :] = j + b * V
        d = jnp.where(iota == j[:, None], jnp.inf, d)


def _knn(x, V, K):
    body = functools.partial(_knn_body, V=V, K=K)
    return pl.pallas_call(
        body,
        grid=(B,),
        in_specs=[pl.BlockSpec((1, V, 3), lambda b: (b, 0, 0))],
        out_specs=pl.BlockSpec((1, 16, V), lambda b: (b, 0, 0)),
        out_shape=jax.ShapeDtypeStruct((B, 16, V), jnp.int32),
    )(x)


NW = 32


@functools.lru_cache(maxsize=None)
def _sc_gather_call(R, D, M):
    chunk = M // NW
    G = 128 if chunk % 128 == 0 else 64
    rounds = chunk // G
    mesh = plsc.VectorSubcoreMesh(core_axis_name="c", subcore_axis_name="s")

    @functools.partial(
        pl.kernel,
        out_type=jax.ShapeDtypeStruct((M, D), F32),
        mesh=mesh,
        scratch_types=[
            pltpu.VMEM((chunk,), jnp.int32),
            pltpu.VMEM((G, D), F32),
            pltpu.SemaphoreType.DMA,
        ],
    )
    def k(table_hbm, idx_hbm, out_hbm, idx_v, rows_v, sem):
        wid = lax.axis_index("s") * 2 + lax.axis_index("c")
        base = wid * chunk
        pltpu.sync_copy(idx_hbm.at[pl.ds(base, chunk)], idx_v)
        for i in range(rounds):
            pltpu.async_copy(table_hbm.at[idx_v.at[pl.ds(i * G, G)]],
                             rows_v, sem).wait()
            pltpu.sync_copy(rows_v, out_hbm.at[pl.ds(base + i * G, G)])

    return k


def _gather_rows(table, idx):
    R, D = table.shape
    (M,) = idx.shape
    return _sc_gather_call(R, D, M)(table, idx)


def _surf_body(x_ref, xg_ref, d0_ref, out_ref, *, V, N, C, COFF):
    x = x_ref[0]
    sdn = _normcols(d0_ref[...])
    acc = jnp.full((V, S * C), -jnp.inf, F32)
    for n in range(N):
        xj = xg_ref[0, n, :, COFF:COFF + 3]
        dirv = xj - x
        nrm = jnp.sqrt(jnp.sum(dirv * dirv, axis=1, keepdims=True))
        ndn = dirv / jnp.maximum(nrm, 1e-12)
        th = jnp.dot(ndn, sdn, preferred_element_type=F32)
        acc = jnp.maximum(acc, _silu(th))
    s = acc[:, 0:C]
    for si in range(1, S):
        s = s + acc[:, si * C:(si + 1) * C]
    out_ref[0] = _silu(s)


def _conv_surface(x, xg, d0, V, N, C, COFF=0):
    body = functools.partial(_surf_body, V=V, N=N, C=C, COFF=COFF)
    Wc = xg.shape[-1]
    return pl.pallas_call(
        body,
        grid=(B,),
        in_specs=[
            pl.BlockSpec((1, V, 3), lambda b: (b, 0, 0)),
            pl.BlockSpec((1, N, V, Wc), lambda b: (b, 0, 0, 0)),
            pl.BlockSpec((3, S * C), lambda b: (0, 0)),
        ],
        out_specs=pl.BlockSpec((1, V, C), lambda b: (b, 0, 0)),
        out_shape=jax.ShapeDtypeStruct((B, V, C), F32),
    )(x, xg, d0)


def _conv_body(x_ref, fm_ref, g_ref, xg_ref, w_ref, b_ref, d_ref, out_ref,
               *, V, N, Cin, Cout, COFF):
    x = x_ref[0]
    fm = fm_ref[0]
    w = w_ref[...].astype(jnp.bfloat16)
    bias = b_ref[...]
    sdn = _normcols(d_ref[...])
    center = (jnp.dot(fm.astype(jnp.bfloat16), w[:, 0:Cout],
                      preferred_element_type=F32) + bias[:, 0:Cout])
    acc = jnp.full((V, S * Cout), -jnp.inf, F32)
    for n in range(N):
        gn = g_ref[0, n, :, 0:Cin].astype(jnp.bfloat16)
        sup = (jnp.dot(gn, w[:, Cout:], preferred_element_type=F32)
               + bias[:, Cout:])
        xj = xg_ref[0, n, :, COFF:COFF + 3]
        dirv = xj - x
        nrm = jnp.sqrt(jnp.sum(dirv * dirv, axis=1, keepdims=True))
        ndn = dirv / jnp.maximum(nrm, 1e-12)
        th = _silu(jnp.dot(ndn, sdn, preferred_element_type=F32))
        acc = jnp.maximum(acc, th * sup)
    act = acc[:, 0:Cout]
    for si in range(1, S):
        act = act + acc[:, si * Cout:(si + 1) * Cout]
    out_ref[0] = center + act


def _conv_layer(x, fm, g, xg, w, bias, d, V, N, Cin, Cout, COFF=0):
    body = functools.partial(_conv_body, V=V, N=N, Cin=Cin, Cout=Cout,
                             COFF=COFF)
    wtot = (S + 1) * Cout
    Wf = g.shape[-1]
    Wc = xg.shape[-1]
    return pl.pallas_call(
        body,
        grid=(B,),
        in_specs=[
            pl.BlockSpec((1, V, 3), lambda b: (b, 0, 0)),
            pl.BlockSpec((1, V, Cin), lambda b: (b, 0, 0)),
            pl.BlockSpec((1, N, V, Wf), lambda b: (b, 0, 0, 0)),
            pl.BlockSpec((1, N, V, Wc), lambda b: (b, 0, 0, 0)),
            pl.BlockSpec((Cin, wtot), lambda b: (0, 0)),
            pl.BlockSpec((1, wtot), lambda b: (0, 0)),
            pl.BlockSpec((3, S * Cout), lambda b: (0, 0)),
        ],
        out_specs=pl.BlockSpec((1, V, Cout), lambda b: (b, 0, 0)),
        out_shape=jax.ShapeDtypeStruct((B, V, Cout), F32),
    )(x, fm, g, xg, w, bias.reshape(1, -1), d)


def _bn_body(y_ref, g_ref, be_ref, out_ref, *, V, C):
    y = y_ref[...].reshape(B * V, C)
    mean = jnp.mean(y, axis=0, keepdims=True)
    var = jnp.mean((y - mean) ** 2, axis=0, keepdims=True)
    xn = (y - mean) / jnp.sqrt(var + 1e-5) * g_ref[...] + be_ref[...]
    out_ref[...] = _silu(xn).reshape(B, V, C)


def _bn_silu(y, gamma, beta, V, C):
    body = functools.partial(_bn_body, V=V, C=C)
    return pl.pallas_call(
        body,
        grid=(1,),
        in_specs=[
            pl.BlockSpec((B, V, C), lambda i: (0, 0, 0)),
            pl.BlockSpec((1, C), lambda i: (0, 0)),
            pl.BlockSpec((1, C), lambda i: (0, 0)),
        ],
        out_specs=pl.BlockSpec((B, V, C), lambda i: (0, 0, 0)),
        out_shape=jax.ShapeDtypeStruct((B, V, C), F32),
    )(y, gamma.reshape(1, -1), beta.reshape(1, -1))


def _pool_body(x_ref, fm_ref, idx_ref, samp_ref, vout_ref, fout_ref,
               *, V, C, SS):
    b = pl.program_id(0)
    si = samp_ref[...].reshape(SS, 1)
    iota = lax.broadcasted_iota(jnp.int32, (SS, V), 1)
    P = (si == iota)
    x = x_ref[0]
    vcols = []
    for c in range(3):
        col = jnp.sum(jnp.where(P, x[:, c][None, :], 0.0), axis=1)
        vcols.append(col[:, None])
    vout_ref[0] = jnp.concatenate(vcols, axis=1)
    fm = fm_ref[0]
    acc = jnp.full((SS, C), -jnp.inf, F32)
    for m in range(4):
        row = idx_ref[0, m, :] - b * V
        cm = jnp.sum(jnp.where(P, row[None, :], 0), axis=1)
        Q = (cm[:, None] == iota).astype(F32)
        acc = jnp.maximum(acc, jnp.dot(Q, fm, preferred_element_type=F32))
    fout_ref[0] = acc


def _pool(x, fm, idxT, samp, V, C, SS):
    body = functools.partial(_pool_body, V=V, C=C, SS=SS)
    return pl.pallas_call(
        body,
        grid=(B,),
        in_specs=[
            pl.BlockSpec((1, V, 3), lambda b: (b, 0, 0)),
            pl.BlockSpec((1, V, C), lambda b: (b, 0, 0)),
            pl.BlockSpec((1, 16, V), lambda b: (b, 0, 0)),
            pl.BlockSpec((1, SS), lambda b: (0, 0)),
        ],
        out_specs=[
            pl.BlockSpec((1, SS, 3), lambda b: (b, 0, 0)),
            pl.BlockSpec((1, SS, C), lambda b: (b, 0, 0)),
        ],
        out_shape=[
            jax.ShapeDtypeStruct((B, SS, 3), F32),
            jax.ShapeDtypeStruct((B, SS, C), F32),
        ],
    )(x, fm, idxT, samp.reshape(1, SS).astype(jnp.int32))


def kernel(x, d0, w1, b1, d1, g1, be1, w2, b2, d2, g2, be2, w3, b3, d3, g3,
           be3, w4, b4, d4, sample_idx1, sample_idx2):
    idx1 = _knn(x, V1, NBR)
    nb1 = idx1[:, :NBR, :].reshape(-1)
    x_pad = jnp.pad(x.reshape(B * V1, 3), ((0, 0), (0, 125)))
    xg1 = _gather_rows(x_pad, nb1).reshape(B, NBR, V1, 128)
    fm0 = _conv_surface(x, xg1, d0, V1, NBR, 128)
    g0 = _gather_rows(fm0.reshape(B * V1, 128), nb1).reshape(B, NBR, V1, 128)
    y1 = _conv_layer(x, fm0, g0, xg1, w1, b1, d1, V1, NBR, 128, 128)
    fm1 = _bn_silu(y1, g1, be1, V1, 128)
    v1, fmp1 = _pool(x, fm1, idx1, sample_idx1, V1, 128, V2)

    idx2 = _knn(v1, V2, NBR)
    nb2 = idx2[:, :NBR, :].reshape(-1)
    v1_pad = jnp.pad(v1.reshape(B * V2, 3), ((0, 0), (0, 125)))
    t2 = jnp.concatenate([fmp1.reshape(B * V2, 128), v1_pad], axis=1)
    g12 = _gather_rows(t2, nb2).reshape(B, NBR, V2, 256)
    y2 = _conv_layer(v1, fmp1, g12, g12, w2, b2, d2, V2, NBR, 128, 256,
                     COFF=128)
    fm2 = _bn_silu(y2, g2, be2, V2, 256)
    g2g = _gather_rows(fm2.reshape(B * V2, 256), nb2).reshape(B, NBR, V2, 256)
    y3 = _conv_layer(v1, fm2, g2g, g12, w3, b3, d3, V2, NBR, 256, 256,
                     COFF=128)
    fm3 = _bn_silu(y3, g3, be3, V2, 256)
    v2, fmp2 = _pool(v1, fm3, idx2, sample_idx2, V2, 256, V3)

    n3 = min(NBR, V3 // 8)
    idx3 = _knn(v2, V3, n3)
    nb3 = idx3[:, :n3, :].reshape(-1)
    v2_pad = jnp.pad(v2.reshape(B * V3, 3), ((0, 0), (0, 125)))
    t3 = jnp.concatenate([fmp2.reshape(B * V3, 256), v2_pad], axis=1)
    g34 = _gather_rows(t3, nb3).reshape(B, n3, V3, 384)
    fm4 = _conv_layer(v2, fmp2, g34, g34, w4, b4, d4, V3, n3, 256, 512,
                      COFF=256)
    return fm4


# speedup vs baseline: 1.0332x; 1.0332x over previous
import functools

import jax
import jax.numpy as jnp
from jax import lax
from jax.experimental import pallas as pl
from jax.experimental.pallas import tpu as pltpu
from jax.experimental.pallas import tpu_sc as plsc

B = 4
V1 = 1024
V2 = 256
V3 = 64
S = 7
NBR = 10
F32 = jnp.float32


def _silu(x):
    return x / (1.0 + jnp.exp(-x))


def _normcols(d):
    n = jnp.sqrt(jnp.sum(d * d, axis=0, keepdims=True))
    return d / jnp.maximum(n, 1e-12)


def _knn_body(x_ref, out_ref, *, V, K):
    b = pl.program_id(0)
    x = x_ref[0]
    q = jnp.sum(x * x, axis=1)
    d = (q[:, None] + q[None, :]
         - 2.0 * lax.dot_general(x, x, (((1,), (1,)), ((), ())),
                                 preferred_element_type=F32))
    iota = lax.broadcasted_iota(jnp.int32, (V, V), 1)
    out_ref[0] = jnp.zeros((16, V), jnp.int32)
    for p in range(K + 1):
        m = jnp.min(d, axis=1)
        eq = d == m[:, None]
        j = jnp.min(jnp.where(eq, iota, V), axis=1)
        if p > 0:
            out_ref[0, p - 1, :] = j + b * V
        d = jnp.where(iota == j[:, None], jnp.inf, d)


def _knn(x, V, K):
    body = functools.partial(_knn_body, V=V, K=K)
    return pl.pallas_call(
        body,
        grid=(B,),
        in_specs=[pl.BlockSpec((1, V, 3), lambda b: (b, 0, 0))],
        out_specs=pl.BlockSpec((1, 16, V), lambda b: (b, 0, 0)),
        out_shape=jax.ShapeDtypeStruct((B, 16, V), jnp.int32),
    )(x)


NW = 32


@functools.lru_cache(maxsize=None)
def _sc_gather_call(R, D, M):
    chunk = M // NW
    G = 128 if chunk % 128 == 0 else 64
    rounds = chunk // G
    mesh = plsc.VectorSubcoreMesh(core_axis_name="c", subcore_axis_name="s")

    @functools.partial(
        pl.kernel,
        out_type=jax.ShapeDtypeStruct((M, D), F32),
        mesh=mesh,
        scratch_types=[
            pltpu.VMEM((chunk,), jnp.int32),
            pltpu.VMEM((2, G, D), F32),
            pltpu.SemaphoreType.DMA,
            pltpu.SemaphoreType.DMA,
            pltpu.SemaphoreType.DMA,
            pltpu.SemaphoreType.DMA,
        ],
    )
    def k(table_hbm, idx_hbm, out_hbm, idx_v, rows_v, g0, g1, w0, w1):
        wid = lax.axis_index("s") * 2 + lax.axis_index("c")
        base = wid * chunk
        gsems = (g0, g1)
        wsems = (w0, w1)
        pltpu.sync_copy(idx_hbm.at[pl.ds(base, chunk)], idx_v)
        gd = {}
        wd = {0: None, 1: None}

        def start_gather(i):
            p = i % 2
            gd[p] = pltpu.async_copy(
                table_hbm.at[idx_v.at[pl.ds(i * G, G)]], rows_v.at[p],
                gsems[p])

        start_gather(0)
        for i in range(rounds):
            p = i % 2
            if i + 1 < rounds:
                q = (i + 1) % 2
                if wd[q] is not None:
                    wd[q].wait()
                    wd[q] = None
                start_gather(i + 1)
            gd[p].wait()
            wd[p] = pltpu.async_copy(
                rows_v.at[p], out_hbm.at[pl.ds(base + i * G, G)], wsems[p])
        for p in (0, 1):
            if wd[p] is not None:
                wd[p].wait()

    return k


def _gather_rows(table, idx):
    R, D = table.shape
    (M,) = idx.shape
    return _sc_gather_call(R, D, M)(table, idx)


def _surf_body(x_ref, xg_ref, d0_ref, out_ref, *, V, N, C, COFF):
    x = x_ref[0]
    sdn = _normcols(d0_ref[...])
    acc = jnp.full((V, S * C), -jnp.inf, F32)
    for n in range(N):
        xj = xg_ref[0, n, :, COFF:COFF + 3]
        dirv = xj - x
        nrm = jnp.sqrt(jnp.sum(dirv * dirv, axis=1, keepdims=True))
        ndn = dirv / jnp.maximum(nrm, 1e-12)
        th = jnp.dot(ndn, sdn, preferred_element_type=F32)
        acc = jnp.maximum(acc, _silu(th))
    s = acc[:, 0:C]
    for si in range(1, S):
        s = s + acc[:, si * C:(si + 1) * C]
    out_ref[0] = _silu(s)


def _conv_surface(x, xg, d0, V, N, C, COFF=0):
    body = functools.partial(_surf_body, V=V, N=N, C=C, COFF=COFF)
    Wc = xg.shape[-1]
    return pl.pallas_call(
        body,
        grid=(B,),
        in_specs=[
            pl.BlockSpec((1, V, 3), lambda b: (b, 0, 0)),
            pl.BlockSpec((1, N, V, Wc), lambda b: (b, 0, 0, 0)),
            pl.BlockSpec((3, S * C), lambda b: (0, 0)),
        ],
        out_specs=pl.BlockSpec((1, V, C), lambda b: (b, 0, 0)),
        out_shape=jax.ShapeDtypeStruct((B, V, C), F32),
    )(x, xg, d0)


def _conv_body(x_ref, fm_ref, g_ref, xg_ref, w_ref, b_ref, d_ref, out_ref,
               *, V, N, Cin, Cout, COFF):
    x = x_ref[0]
    fm = fm_ref[0]
    w = w_ref[...].astype(jnp.bfloat16)
    bias = b_ref[...]
    sdn = _normcols(d_ref[...])
    center = (jnp.dot(fm.astype(jnp.bfloat16), w[:, 0:Cout],
                      preferred_element_type=F32) + bias[:, 0:Cout])
    acc = jnp.full((V, S * Cout), -jnp.inf, F32)
    for n in range(N):
        gn = g_ref[0, n, :, 0:Cin].astype(jnp.bfloat16)
        sup = (jnp.dot(gn, w[:, Cout:], preferred_element_type=F32)
               + bias[:, Cout:])
        xj = xg_ref[0, n, :, COFF:COFF + 3]
        dirv = xj - x
        nrm = jnp.sqrt(jnp.sum(dirv * dirv, axis=1, keepdims=True))
        ndn = dirv / jnp.maximum(nrm, 1e-12)
        th = _silu(jnp.dot(ndn, sdn, preferred_element_type=F32))
        acc = jnp.maximum(acc, th * sup)
    act = acc[:, 0:Cout]
    for si in range(1, S):
        act = act + acc[:, si * Cout:(si + 1) * Cout]
    out_ref[0] = center + act


def _conv_layer(x, fm, g, xg, w, bias, d, V, N, Cin, Cout, COFF=0):
    body = functools.partial(_conv_body, V=V, N=N, Cin=Cin, Cout=Cout,
                             COFF=COFF)
    wtot = (S + 1) * Cout
    Wf = g.shape[-1]
    Wc = xg.shape[-1]
    return pl.pallas_call(
        body,
        grid=(B,),
        in_specs=[
            pl.BlockSpec((1, V, 3), lambda b: (b, 0, 0)),
            pl.BlockSpec((1, V, Cin), lambda b: (b, 0, 0)),
            pl.BlockSpec((1, N, V, Wf), lambda b: (b, 0, 0, 0)),
            pl.BlockSpec((1, N, V, Wc), lambda b: (b, 0, 0, 0)),
            pl.BlockSpec((Cin, wtot), lambda b: (0, 0)),
            pl.BlockSpec((1, wtot), lambda b: (0, 0)),
            pl.BlockSpec((3, S * Cout), lambda b: (0, 0)),
        ],
        out_specs=pl.BlockSpec((1, V, Cout), lambda b: (b, 0, 0)),
        out_shape=jax.ShapeDtypeStruct((B, V, Cout), F32),
    )(x, fm, g, xg, w, bias.reshape(1, -1), d)


def _bn_body(y_ref, g_ref, be_ref, out_ref, *, V, C):
    y = y_ref[...].reshape(B * V, C)
    mean = jnp.mean(y, axis=0, keepdims=True)
    var = jnp.mean((y - mean) ** 2, axis=0, keepdims=True)
    xn = (y - mean) / jnp.sqrt(var + 1e-5) * g_ref[...] + be_ref[...]
    out_ref[...] = _silu(xn).reshape(B, V, C)


def _bn_silu(y, gamma, beta, V, C):
    body = functools.partial(_bn_body, V=V, C=C)
    return pl.pallas_call(
        body,
        grid=(1,),
        in_specs=[
            pl.BlockSpec((B, V, C), lambda i: (0, 0, 0)),
            pl.BlockSpec((1, C), lambda i: (0, 0)),
            pl.BlockSpec((1, C), lambda i: (0, 0)),
        ],
        out_specs=pl.BlockSpec((B, V, C), lambda i: (0, 0, 0)),
        out_shape=jax.ShapeDtypeStruct((B, V, C), F32),
    )(y, gamma.reshape(1, -1), beta.reshape(1, -1))


def _pool_body(x_ref, fm_ref, idx_ref, samp_ref, vout_ref, fout_ref,
               *, V, C, SS):
    b = pl.program_id(0)
    si = samp_ref[...].reshape(SS, 1)
    iota = lax.broadcasted_iota(jnp.int32, (SS, V), 1)
    P = (si == iota)
    x = x_ref[0]
    vcols = []
    for c in range(3):
        col = jnp.sum(jnp.where(P, x[:, c][None, :], 0.0), axis=1)
        vcols.append(col[:, None])
    vout_ref[0] = jnp.concatenate(vcols, axis=1)
    fm = fm_ref[0]
    acc = jnp.full((SS, C), -jnp.inf, F32)
    for m in range(4):
        row = idx_ref[0, m, :] - b * V
        cm = jnp.sum(jnp.where(P, row[None, :], 0), axis=1)
        Q = (cm[:, None] == iota).astype(F32)
        acc = jnp.maximum(acc, jnp.dot(Q, fm, preferred_element_type=F32))
    fout_ref[0] = acc


def _pool(x, fm, idxT, samp, V, C, SS):
    body = functools.partial(_pool_body, V=V, C=C, SS=SS)
    return pl.pallas_call(
        body,
        grid=(B,),
        in_specs=[
            pl.BlockSpec((1, V, 3), lambda b: (b, 0, 0)),
            pl.BlockSpec((1, V, C), lambda b: (b, 0, 0)),
            pl.BlockSpec((1, 16, V), lambda b: (b, 0, 0)),
            pl.BlockSpec((1, SS), lambda b: (0, 0)),
        ],
        out_specs=[
            pl.BlockSpec((1, SS, 3), lambda b: (b, 0, 0)),
            pl.BlockSpec((1, SS, C), lambda b: (b, 0, 0)),
        ],
        out_shape=[
            jax.ShapeDtypeStruct((B, SS, 3), F32),
            jax.ShapeDtypeStruct((B, SS, C), F32),
        ],
    )(x, fm, idxT, samp.reshape(1, SS).astype(jnp.int32))


def kernel(x, d0, w1, b1, d1, g1, be1, w2, b2, d2, g2, be2, w3, b3, d3, g3,
           be3, w4, b4, d4, sample_idx1, sample_idx2):
    idx1 = _knn(x, V1, NBR)
    nb1 = idx1[:, :NBR, :].reshape(-1)
    x_pad = jnp.pad(x.reshape(B * V1, 3), ((0, 0), (0, 125)))
    xg1 = _gather_rows(x_pad, nb1).reshape(B, NBR, V1, 128)
    fm0 = _conv_surface(x, xg1, d0, V1, NBR, 128)
    g0 = _gather_rows(fm0.reshape(B * V1, 128), nb1).reshape(B, NBR, V1, 128)
    y1 = _conv_layer(x, fm0, g0, xg1, w1, b1, d1, V1, NBR, 128, 128)
    fm1 = _bn_silu(y1, g1, be1, V1, 128)
    v1, fmp1 = _pool(x, fm1, idx1, sample_idx1, V1, 128, V2)

    idx2 = _knn(v1, V2, NBR)
    nb2 = idx2[:, :NBR, :].reshape(-1)
    v1_pad = jnp.pad(v1.reshape(B * V2, 3), ((0, 0), (0, 125)))
    t2 = jnp.concatenate([fmp1.reshape(B * V2, 128), v1_pad], axis=1)
    g12 = _gather_rows(t2, nb2).reshape(B, NBR, V2, 256)
    y2 = _conv_layer(v1, fmp1, g12, g12, w2, b2, d2, V2, NBR, 128, 256,
                     COFF=128)
    fm2 = _bn_silu(y2, g2, be2, V2, 256)
    g2g = _gather_rows(fm2.reshape(B * V2, 256), nb2).reshape(B, NBR, V2, 256)
    y3 = _conv_layer(v1, fm2, g2g, g12, w3, b3, d3, V2, NBR, 256, 256,
                     COFF=128)
    fm3 = _bn_silu(y3, g3, be3, V2, 256)
    v2, fmp2 = _pool(v1, fm3, idx2, sample_idx2, V2, 256, V3)

    n3 = min(NBR, V3 // 8)
    idx3 = _knn(v2, V3, n3)
    nb3 = idx3[:, :n3, :].reshape(-1)
    v2_pad = jnp.pad(v2.reshape(B * V3, 3), ((0, 0), (0, 125)))
    t3 = jnp.concatenate([fmp2.reshape(B * V3, 256), v2_pad], axis=1)
    g34 = _gather_rows(t3, nb3).reshape(B, n3, V3, 384)
    fm4 = _conv_layer(v2, fmp2, g34, g34, w4, b4, d4, V3, n3, 256, 512,
                      COFF=256)
    return fm4


# speedup vs baseline: 1.1788x; 1.1409x over previous
import functools

import jax
import jax.numpy as jnp
from jax import lax
from jax.experimental import pallas as pl
from jax.experimental.pallas import tpu as pltpu
from jax.experimental.pallas import tpu_sc as plsc

B = 4
V1 = 1024
V2 = 256
V3 = 64
S = 7
NBR = 10
F32 = jnp.float32
BF16 = jnp.bfloat16


def _silu(x):
    return x / (1.0 + jnp.exp(-x))


def _normcols(d):
    n = jnp.sqrt(jnp.sum(d * d, axis=0, keepdims=True))
    return d / jnp.maximum(n, 1e-12)


def _knn_body_feat(x_ref, fm_ref, out_ref, xg_ref, gf_ref, *, V, K, C):
    b = pl.program_id(0)
    x = x_ref[0]
    q = jnp.sum(x * x, axis=1)
    d = (q[:, None] + q[None, :]
         - 2.0 * lax.dot_general(x, x, (((1,), (1,)), ((), ())),
                                 preferred_element_type=F32))
    iota = lax.broadcasted_iota(jnp.int32, (V, V), 1)
    out_ref[0] = jnp.zeros((16, V), jnp.int32)
    fm = None if fm_ref is None else fm_ref[0]
    for p in range(K + 1):
        m = jnp.min(d, axis=1)
        eq = d == m[:, None]
        j = jnp.min(jnp.where(eq, iota, V), axis=1)
        if p > 0:
            out_ref[0, p - 1, :] = j + b * V
            sel = eq.astype(F32)
            xg_ref[0, p - 1] = jnp.dot(sel, x, preferred_element_type=F32)
            if fm is not None:
                gf_ref[0, p - 1] = jnp.dot(sel, fm,
                                           preferred_element_type=F32)
        d = jnp.where(eq, jnp.inf, d)


def _knn(x, V, K, fm=None, C=0):
    body = functools.partial(_knn_body_feat, V=V, K=K, C=C)
    in_specs = [pl.BlockSpec((1, V, 3), lambda b: (b, 0, 0))]
    out_specs = [
        pl.BlockSpec((1, 16, V), lambda b: (b, 0, 0)),
        pl.BlockSpec((1, 16, V, 3), lambda b: (b, 0, 0, 0)),
    ]
    out_shape = [
        jax.ShapeDtypeStruct((B, 16, V), jnp.int32),
        jax.ShapeDtypeStruct((B, 16, V, 3), F32),
    ]
    if fm is not None:
        in_specs.append(pl.BlockSpec((1, V, C), lambda b: (b, 0, 0)))
        out_specs.append(pl.BlockSpec((1, 16, V, C), lambda b: (b, 0, 0, 0)))
        out_shape.append(jax.ShapeDtypeStruct((B, 16, V, C), F32))

        def body2(x_ref, fm_ref, out_ref, xg_ref, gf_ref):
            body(x_ref, fm_ref, out_ref, xg_ref, gf_ref)

        return pl.pallas_call(body2, grid=(B,), in_specs=in_specs,
                              out_specs=out_specs, out_shape=out_shape)(x, fm)

    def body1(x_ref, out_ref, xg_ref):
        body(x_ref, None, out_ref, xg_ref, None)

    return pl.pallas_call(body1, grid=(B,), in_specs=in_specs,
                          out_specs=out_specs, out_shape=out_shape)(x)


NW = 32


@functools.lru_cache(maxsize=None)
def _sc_gather_call(R, D, M):
    chunk = M // NW
    G = 128 if chunk % 128 == 0 else 64
    rounds = chunk // G
    mesh = plsc.VectorSubcoreMesh(core_axis_name="c", subcore_axis_name="s")

    @functools.partial(
        pl.kernel,
        out_type=jax.ShapeDtypeStruct((M, D), F32),
        mesh=mesh,
        scratch_types=[
            pltpu.VMEM((chunk,), jnp.int32),
            pltpu.VMEM((2, G, D), F32),
            pltpu.SemaphoreType.DMA,
            pltpu.SemaphoreType.DMA,
            pltpu.SemaphoreType.DMA,
            pltpu.SemaphoreType.DMA,
        ],
    )
    def k(table_hbm, idx_hbm, out_hbm, idx_v, rows_v, g0, g1, w0, w1):
        wid = lax.axis_index("s") * 2 + lax.axis_index("c")
        base = wid * chunk
        gsems = (g0, g1)
        wsems = (w0, w1)
        pltpu.sync_copy(idx_hbm.at[pl.ds(base, chunk)], idx_v)
        gd = {}
        wd = {0: None, 1: None}

        def start_gather(i):
            p = i % 2
            gd[p] = pltpu.async_copy(
                table_hbm.at[idx_v.at[pl.ds(i * G, G)]], rows_v.at[p],
                gsems[p])

        start_gather(0)
        for i in range(rounds):
            p = i % 2
            if i + 1 < rounds:
                qq = (i + 1) % 2
                if wd[qq] is not None:
                    wd[qq].wait()
                    wd[qq] = None
                start_gather(i + 1)
            gd[p].wait()
            wd[p] = pltpu.async_copy(
                rows_v.at[p], out_hbm.at[pl.ds(base + i * G, G)], wsems[p])
        for p in (0, 1):
            if wd[p] is not None:
                wd[p].wait()

    return k


def _gather_rows(table, idx):
    R, D = table.shape
    (M,) = idx.shape
    return _sc_gather_call(R, D, M)(table, idx)


def _surf_body(x_ref, xg_ref, d0_ref, out_ref, *, V, N, C):
    x = x_ref[0]
    sdn = _normcols(d0_ref[...])
    acc = jnp.full((V, S * C), -jnp.inf, F32)
    for n in range(N):
        xj = xg_ref[0, n]
        dirv = xj - x
        nrm = jnp.sqrt(jnp.sum(dirv * dirv, axis=1, keepdims=True))
        ndn = dirv / jnp.maximum(nrm, 1e-12)
        th = jnp.dot(ndn, sdn, preferred_element_type=F32)
        acc = jnp.maximum(acc, _silu(th))
    s = acc[:, 0:C]
    for si in range(1, S):
        s = s + acc[:, si * C:(si + 1) * C]
    out_ref[0] = _silu(s)


def _conv_surface(x, xg, d0, V, N, C):
    body = functools.partial(_surf_body, V=V, N=N, C=C)
    return pl.pallas_call(
        body,
        grid=(B,),
        in_specs=[
            pl.BlockSpec((1, V, 3), lambda b: (b, 0, 0)),
            pl.BlockSpec((1, 16, V, 3), lambda b: (b, 0, 0, 0)),
            pl.BlockSpec((3, S * C), lambda b: (0, 0)),
        ],
        out_specs=pl.BlockSpec((1, V, C), lambda b: (b, 0, 0)),
        out_shape=jax.ShapeDtypeStruct((B, V, C), F32),
    )(x, xg, d0)


def _conv_body(x_ref, fm_ref, g_ref, xg_ref, w_ref, b_ref, d_ref, out_ref,
               ps_ref, *, V, N, Cin, Cout):
    x = x_ref[0]
    fm = fm_ref[0]
    w = w_ref[...].astype(BF16)
    bias = b_ref[...]
    sdn = _normcols(d_ref[...])
    center = (jnp.dot(fm.astype(BF16), w[:, 0:Cout],
                      preferred_element_type=F32) + bias[:, 0:Cout])
    acc = jnp.full((V, S * Cout), -jnp.inf, F32)
    for n in range(N):
        gn = g_ref[0, n, :, 0:Cin].astype(BF16)
        sup = (jnp.dot(gn, w[:, Cout:], preferred_element_type=F32)
               + bias[:, Cout:])
        xj = xg_ref[0, n]
        dirv = xj - x
        nrm = jnp.sqrt(jnp.sum(dirv * dirv, axis=1, keepdims=True))
        ndn = dirv / jnp.maximum(nrm, 1e-12)
        th = _silu(jnp.dot(ndn, sdn, preferred_element_type=F32))
        acc = jnp.maximum(acc, th * sup)
    act = acc[:, 0:Cout]
    for si in range(1, S):
        act = act + acc[:, si * Cout:(si + 1) * Cout]
    y = center + act
    out_ref[0] = y
    if ps_ref is not None:
        ps_ref[0] = jnp.concatenate(
            [jnp.sum(y, axis=0)[None, :], jnp.sum(y * y, axis=0)[None, :],
             jnp.zeros((6, Cout), F32)], axis=0)


def _conv_layer(x, fm, g, xg, w, bias, d, V, N, Cin, Cout, stats=True):
    wtot = (S + 1) * Cout
    NG = g.shape[1]
    in_specs = [
        pl.BlockSpec((1, V, 3), lambda b: (b, 0, 0)),
        pl.BlockSpec((1, V, Cin), lambda b: (b, 0, 0)),
        pl.BlockSpec((1, NG, V, g.shape[-1]), lambda b: (b, 0, 0, 0)),
        pl.BlockSpec((1, 16, V, 3), lambda b: (b, 0, 0, 0)),
        pl.BlockSpec((Cin, wtot), lambda b: (0, 0)),
        pl.BlockSpec((1, wtot), lambda b: (0, 0)),
        pl.BlockSpec((3, S * Cout), lambda b: (0, 0)),
    ]
    if stats:
        body = functools.partial(_conv_body, V=V, N=N, Cin=Cin, Cout=Cout)
        return pl.pallas_call(
            body,
            grid=(B,),
            in_specs=in_specs,
            out_specs=[
                pl.BlockSpec((1, V, Cout), lambda b: (b, 0, 0)),
                pl.BlockSpec((1, 8, Cout), lambda b: (b, 0, 0)),
            ],
            out_shape=[
                jax.ShapeDtypeStruct((B, V, Cout), F32),
                jax.ShapeDtypeStruct((B, 8, Cout), F32),
            ],
        )(x, fm, g, xg, w, bias.reshape(1, -1), d)

    def body_ns(x_ref, fm_ref, g_ref, xg_ref, w_ref, b_ref, d_ref, out_ref):
        _conv_body(x_ref, fm_ref, g_ref, xg_ref, w_ref, b_ref, d_ref,
                   out_ref, None, V=V, N=N, Cin=Cin, Cout=Cout)

    return pl.pallas_call(
        body_ns,
        grid=(B,),
        in_specs=in_specs,
        out_specs=pl.BlockSpec((1, V, Cout), lambda b: (b, 0, 0)),
        out_shape=jax.ShapeDtypeStruct((B, V, Cout), F32),
    )(x, fm, g, xg, w, bias.reshape(1, -1), d)


def _bn_scale_shift(ps, g_ref, be_ref, V):
    tot = jnp.sum(ps[:, 0, :], axis=0) / (B * V)
    tot2 = jnp.sum(ps[:, 1, :], axis=0) / (B * V)
    var = tot2 - tot * tot
    sc = g_ref[...][0] / jnp.sqrt(var + 1e-5)
    sh = be_ref[...][0] - tot * sc
    return sc, sh


def _bn_body(y_ref, ps_ref, g_ref, be_ref, out_ref, *, V, C):
    sc, sh = _bn_scale_shift(ps_ref[...], g_ref, be_ref, V)
    y = y_ref[...].reshape(B * V, C)
    out_ref[...] = _silu(y * sc[None, :] + sh[None, :]).reshape(B, V, C)


def _bn_silu(y, ps, gamma, beta, V, C):
    body = functools.partial(_bn_body, V=V, C=C)
    return pl.pallas_call(
        body,
        grid=(1,),
        in_specs=[
            pl.BlockSpec((B, V, C), lambda i: (0, 0, 0)),
            pl.BlockSpec((B, 8, C), lambda i: (0, 0, 0)),
            pl.BlockSpec((1, C), lambda i: (0, 0)),
            pl.BlockSpec((1, C), lambda i: (0, 0)),
        ],
        out_specs=pl.BlockSpec((B, V, C), lambda i: (0, 0, 0)),
        out_shape=jax.ShapeDtypeStruct((B, V, C), F32),
    )(y, ps, gamma.reshape(1, -1), beta.reshape(1, -1))


def _pool_body(x_ref, y_ref, ps_ref, g_ref, be_ref, idx_ref, samp_ref,
               vout_ref, fout_ref, *, V, C, SS):
    b = pl.program_id(0)
    sc, sh = _bn_scale_shift(ps_ref[...], g_ref, be_ref, V)
    fm = _silu(y_ref[0] * sc[None, :] + sh[None, :])
    si = samp_ref[...].reshape(SS, 1)
    iota = lax.broadcasted_iota(jnp.int32, (SS, V), 1)
    P = (si == iota)
    x = x_ref[0]
    vcols = []
    for c in range(3):
        col = jnp.sum(jnp.where(P, x[:, c][None, :], 0.0), axis=1)
        vcols.append(col[:, None])
    vout_ref[0] = jnp.concatenate(vcols, axis=1)
    acc = jnp.full((SS, C), -jnp.inf, F32)
    for m in range(4):
        row = idx_ref[0, m, :] - b * V
        cm = jnp.sum(jnp.where(P, row[None, :], 0), axis=1)
        Q = (cm[:, None] == iota).astype(F32)
        acc = jnp.maximum(acc, jnp.dot(Q, fm, preferred_element_type=F32))
    fout_ref[0] = acc


def _pool(x, y, ps, gamma, beta, idxT, samp, V, C, SS):
    body = functools.partial(_pool_body, V=V, C=C, SS=SS)
    return pl.pallas_call(
        body,
        grid=(B,),
        in_specs=[
            pl.BlockSpec((1, V, 3), lambda b: (b, 0, 0)),
            pl.BlockSpec((1, V, C), lambda b: (b, 0, 0)),
            pl.BlockSpec((B, 8, C), lambda b: (0, 0, 0)),
            pl.BlockSpec((1, C), lambda b: (0, 0)),
            pl.BlockSpec((1, C), lambda b: (0, 0)),
            pl.BlockSpec((1, 16, V), lambda b: (b, 0, 0)),
            pl.BlockSpec((1, SS), lambda b: (0, 0)),
        ],
        out_specs=[
            pl.BlockSpec((1, SS, 3), lambda b: (b, 0, 0)),
            pl.BlockSpec((1, SS, C), lambda b: (b, 0, 0)),
        ],
        out_shape=[
            jax.ShapeDtypeStruct((B, SS, 3), F32),
            jax.ShapeDtypeStruct((B, SS, C), F32),
        ],
    )(x, y, ps, gamma.reshape(1, -1), beta.reshape(1, -1), idxT,
      samp.reshape(1, SS).astype(jnp.int32))


def kernel(x, d0, w1, b1, d1, g1, be1, w2, b2, d2, g2, be2, w3, b3, d3, g3,
           be3, w4, b4, d4, sample_idx1, sample_idx2):
    idx1, xg1 = _knn(x, V1, NBR)
    nb1 = idx1[:, :NBR, :].reshape(-1)
    fm0 = _conv_surface(x, xg1, d0, V1, NBR, 128)
    g0 = _gather_rows(fm0.reshape(B * V1, 128), nb1).reshape(B, NBR, V1, 128)
    y1, ps1 = _conv_layer(x, fm0, g0, xg1, w1, b1, d1, V1, NBR, 128, 128)
    v1, fmp1 = _pool(x, y1, ps1, g1, be1, idx1, sample_idx1, V1, 128, V2)

    idx2, xg2, gp1 = _knn(v1, V2, NBR, fm=fmp1, C=128)
    nb2 = idx2[:, :NBR, :].reshape(-1)
    y2, ps2 = _conv_layer(v1, fmp1, gp1, xg2, w2, b2, d2, V2, NBR, 128, 256)
    fm2 = _bn_silu(y2, ps2, g2, be2, V2, 256)
    g2g = _gather_rows(fm2.reshape(B * V2, 256), nb2).reshape(B, NBR, V2, 256)
    y3, ps3 = _conv_layer(v1, fm2, g2g, xg2, w3, b3, d3, V2, NBR, 256, 256)
    v2, fmp2 = _pool(v1, y3, ps3, g3, be3, idx2, sample_idx2, V2, 256, V3)

    n3 = min(NBR, V3 // 8)
    idx3, xg3, g34 = _knn(v2, V3, n3, fm=fmp2, C=256)
    fm4 = _conv_layer(v2, fmp2, g34, xg3, w4, b4, d4, V3, n3, 256, 512,
                      stats=False)
    return fm4
